# baseline (speedup 1.0000x reference)
import math

import jax
import jax.numpy as jnp
from jax import lax
from jax.experimental import pallas as pl
from jax.experimental.pallas import tpu as pltpu
from jax.experimental.pallas import tpu_sc as plsc

N = 10000
E = 320000
D = 128
DOUT = 64
ALPHA = 0.1
THETA = 0.5
NLAYERS = 4

NC = 2
NS = 16
NW = NC * NS
B = 128
C0 = 109
C1 = 49
EPAD = (C0 + C1) * NS * B
NPAD = 10112
RPT = NPAD // NS
DNPAD = 10240
DRPT = DNPAD // NS
BN = 1000

_mesh = plsc.VectorSubcoreMesh(core_axis_name="c", subcore_axis_name="s")


def _deg_body(col_hbm, zeros_hbm, out_hbm, colv, onesv, deg_sh):
    cid = lax.axis_index("c")
    sid = lax.axis_index("s")
    pltpu.sync_copy(zeros_hbm.at[pl.ds(sid * DRPT, DRPT)],
                    deg_sh.at[pl.ds(sid * DRPT, DRPT)])
    pltpu.sync_copy(col_hbm.at[cid, sid], colv)
    for i in range(B // 16):
        onesv[pl.ds(i * 16, 16)] = jnp.full((16,), 1.0, jnp.float32)
    plsc.subcore_barrier()

    def body(j, carry):
        pltpu.sync_copy(onesv, deg_sh.at[colv.at[j]], add=True)
        return carry

    @pl.when(cid == 0)
    def _():
        lax.fori_loop(0, C0, body, 0)

    @pl.when(cid == 1)
    def _():
        lax.fori_loop(0, C1, body, 0)

    plsc.subcore_barrier()
    pltpu.sync_copy(deg_sh.at[pl.ds(sid * DRPT, DRPT)],
                    out_hbm.at[pl.ds(cid * DNPAD + sid * DRPT, DRPT)])


_deg_call = pl.kernel(
    _deg_body,
    out_type=jax.ShapeDtypeStruct((NC * DNPAD,), jnp.float32),
    mesh=_mesh,
    scratch_types=[
        pltpu.VMEM((C0, B), jnp.int32),
        pltpu.VMEM((B,), jnp.float32),
        pltpu.VMEM_SHARED((DNPAD,), jnp.float32),
    ],
)


def _scat_body(hp_hbm, row_hbm, col_hbm, out_hbm,
               ridx, cidx, ibuf, fbuf, s_sh, gsem, ssem, rsem, csem):
    cid = lax.axis_index("c")
    sid = lax.axis_index("s")

    @plsc.parallel_loop(0, B * (D // 16), unroll=8)
    def _zfill(i):
        r = lax.shift_right_logical(i, 3)
        kk = jnp.bitwise_and(i, 7)
        fbuf[0, r, pl.ds(kk * 16, 16)] = jnp.zeros((16,), jnp.float32)

    for q in range(RPT // B):
        pltpu.sync_copy(fbuf.at[0],
                        s_sh.at[pl.ds(sid * RPT + q * B, B)])
    _ztail = RPT - (RPT // B) * B
    if _ztail:
        pltpu.sync_copy(fbuf.at[0, pl.ds(0, _ztail)],
                        s_sh.at[pl.ds(sid * RPT + (RPT // B) * B, _ztail)])
    plsc.subcore_barrier()

    def gather(j, b):
        pltpu.async_copy(hp_hbm.at[ridx.at[b]], ibuf.at[b], gsem)

    def gwait(j, b):
        pltpu.make_async_copy(hp_hbm.at[ridx.at[b]], ibuf.at[b], gsem).wait()

    def rowload(j, s):
        pltpu.async_copy(row_hbm.at[cid, sid, j], ridx.at[s], rsem)

    def rowwait(j, s):
        pltpu.make_async_copy(row_hbm.at[cid, sid, j], ridx.at[s], rsem).wait()

    def colload(j, s):
        pltpu.async_copy(col_hbm.at[cid, sid, j], cidx.at[s], csem)

    def colwait(j, s):
        pltpu.make_async_copy(col_hbm.at[cid, sid, j], cidx.at[s], csem).wait()

    def scat(j, b):
        pltpu.async_copy(fbuf.at[b], s_sh.at[cidx.at[b]], ssem, add=True)

    def swait(j, b):
        pltpu.make_async_copy(fbuf.at[b], s_sh.at[cidx.at[b]], ssem).wait()

    def unpack_chunk(b):
        @plsc.parallel_loop(0, B, unroll=4)
        def urow(r):
            for k in range(4):
                w = ibuf[b, r, pl.ds(k * 16, 16)]
                lo = lax.bitcast_convert_type(
                    jnp.left_shift(w, 16), jnp.float32)
                hi = lax.bitcast_convert_type(
                    jnp.bitwise_and(w, jnp.int32(-65536)), jnp.float32)
                fbuf[b, r, pl.ds(32 * k, 16)] = lo
                fbuf[b, r, pl.ds(32 * k + 16, 16)] = hi

    def pipeline(cn):
        pltpu.sync_copy(row_hbm.at[cid, sid, 0], ridx.at[0])
        gather(0, 0)
        if cn > 1:
            rowload(1, 1)
        colload(0, 0)

        def body(j, carry):
            b = lax.rem(j, 2)
            o = 1 - b

            @pl.when(j > 0)
            def _():
                swait(j - 1, o)

            gwait(j, b)

            @pl.when(j + 1 < cn)
            def _():
                rowwait(j + 1, o)
                gather(j + 1, o)

            @pl.when(j + 2 < cn)
            def _():
                rowload(j + 2, b)

            unpack_chunk(b)
            colwait(j, b)

            @pl.when(j + 1 < cn)
            def _():
                colload(j + 1, o)

            scat(j, b)
            return carry

        lax.fori_loop(0, cn, body, 0)
        last = (cn - 1) % 2
        swait(cn - 1, last)

    @pl.when(cid == 0)
    def _():
        pipeline(C0)

    @pl.when(cid == 1)
    def _():
        pipeline(C1)

    plsc.subcore_barrier()
    pltpu.sync_copy(s_sh.at[pl.ds(sid * RPT, RPT)],
                    out_hbm.at[cid, pl.ds(sid * RPT, RPT)])


_scat_call = pl.kernel(
    _scat_body,
    out_type=jax.ShapeDtypeStruct((NC, NPAD, D), jnp.float32),
    mesh=_mesh,
    compiler_params=pltpu.CompilerParams(use_tc_tiling_on_sc=False),
    scratch_types=[
        pltpu.VMEM((2, B), jnp.int32),
        pltpu.VMEM((2, B), jnp.int32),
        pltpu.VMEM((2, B, D // 2), jnp.int32),
        pltpu.VMEM((2, B, D), jnp.float32),
        pltpu.VMEM_SHARED((NPAD, D), jnp.float32),
        pltpu.SemaphoreType.DMA,
        pltpu.SemaphoreType.DMA,
        pltpu.SemaphoreType.DMA,
        pltpu.SemaphoreType.DMA,
    ],
)


def _stage_a_body(x_ref, w_ref, b_ref, degt_ref, x0_ref, hp_ref, dinv_ref):
    h = jnp.dot(x_ref[...], w_ref[...], preferred_element_type=jnp.float32)
    h = jnp.maximum(h + b_ref[...], 0.0)
    deg = degt_ref[:, 0:1] + degt_ref[:, 1:2] + 1.0
    dinv = lax.rsqrt(deg)
    x0_ref[...] = h
    hp_ref[...] = dinv * h
    dinv_ref[...] = dinv


def _stage_a(x, w0, b0, degt):
    return pl.pallas_call(
        _stage_a_body,
        grid=(N // BN,),
        in_specs=[
            pl.BlockSpec((BN, D), lambda i: (i, 0)),
            pl.BlockSpec((D, D), lambda i: (0, 0)),
            pl.BlockSpec((1, D), lambda i: (0, 0)),
            pl.BlockSpec((BN, NC), lambda i: (i, 0)),
        ],
        out_specs=[
            pl.BlockSpec((BN, D), lambda i: (i, 0)),
            pl.BlockSpec((BN, D), lambda i: (i, 0)),
            pl.BlockSpec((BN, 1), lambda i: (i, 0)),
        ],
        out_shape=[
            jax.ShapeDtypeStruct((N, D), jnp.float32),
            jax.ShapeDtypeStruct((N, D), jnp.float32),
            jax.ShapeDtypeStruct((N, 1), jnp.float32),
        ],
    )(x, w0, b0, degt)


def _layer_common(beta, sp_ref, hp_ref, x0_ref, dinv_ref, w_ref):
    s = sp_ref[0] + sp_ref[1] + hp_ref[...]
    agg = dinv_ref[...] * s
    s = (1.0 - ALPHA) * agg + ALPHA * x0_ref[...]
    t = jnp.dot(s, w_ref[...], preferred_element_type=jnp.float32)
    return (1.0 - beta) * s + beta * t


def _stage_b(sp, hp, x0, dinv, w, beta):
    def body(sp_ref, hp_ref, x0_ref, dinv_ref, w_ref, out_ref):
        t = _layer_common(beta, sp_ref, hp_ref, x0_ref, dinv_ref, w_ref)
        out_ref[...] = dinv_ref[...] * jnp.maximum(t, 0.0)

    return pl.pallas_call(
        body,
        grid=(N // BN,),
        in_specs=[
            pl.BlockSpec((NC, BN, D), lambda i: (0, i, 0)),
            pl.BlockSpec((BN, D), lambda i: (i, 0)),
            pl.BlockSpec((BN, D), lambda i: (i, 0)),
            pl.BlockSpec((BN, 1), lambda i: (i, 0)),
            pl.BlockSpec((D, D), lambda i: (0, 0)),
        ],
        out_specs=pl.BlockSpec((BN, D), lambda i: (i, 0)),
        out_shape=jax.ShapeDtypeStruct((N, D), jnp.float32),
    )(sp, hp, x0, dinv, w)


def _stage_b_final(sp, hp, x0, dinv, w, wlast, blast, beta):
    def body(sp_ref, hp_ref, x0_ref, dinv_ref, w_ref, wl_ref, bl_ref, out_ref):
        t = _layer_common(beta, sp_ref, hp_ref, x0_ref, dinv_ref, w_ref)
        h = jnp.maximum(t, 0.0)
        out_ref[...] = (jnp.dot(h, wl_ref[...], preferred_element_type=jnp.float32)
                        + bl_ref[...])

    return pl.pallas_call(
        body,
        grid=(N // BN,),
        in_specs=[
            pl.BlockSpec((NC, BN, D), lambda i: (0, i, 0)),
            pl.BlockSpec((BN, D), lambda i: (i, 0)),
            pl.BlockSpec((BN, D), lambda i: (i, 0)),
            pl.BlockSpec((BN, 1), lambda i: (i, 0)),
            pl.BlockSpec((D, D), lambda i: (0, 0)),
            pl.BlockSpec((D, DOUT), lambda i: (0, 0)),
            pl.BlockSpec((1, DOUT), lambda i: (0, 0)),
        ],
        out_specs=pl.BlockSpec((BN, DOUT), lambda i: (i, 0)),
        out_shape=jax.ShapeDtypeStruct((N, DOUT), jnp.float32),
    )(sp, hp, x0, dinv, w, wlast, blast)


def kernel(x, edge_index, W0, b0, W1, W2, W3, W4, Wlast, blast):
    row = edge_index[0]
    col = edge_index[1]
    pad = EPAD - E
    e0 = NS * C0 * B
    cpad = NS * C0 * B - (EPAD - e0)

    def _split(a, fill):
        ap = jnp.concatenate([a, jnp.full((pad,), fill, jnp.int32)])
        a0 = ap[:e0].reshape(NS, C0, B)
        a1 = jnp.pad(ap[e0:].reshape(NS, C1, B),
                     ((0, 0), (0, C0 - C1), (0, 0)), constant_values=fill)
        return jnp.stack([a0, a1])

    row3 = _split(row, 0)
    col3 = _split(col, N)
    zeros1 = jnp.zeros((DNPAD,), jnp.float32)

    degp = _deg_call(col3, zeros1).reshape(NC, DNPAD)
    degt = degp[:, :N].T
    x0, hp, dinv = _stage_a(x, W0, b0.reshape(1, D), degt)

    betas = [math.log(THETA / i + 1.0) for i in range(1, NLAYERS + 1)]
    ws = [W1, W2, W3, W4]
    def _pack_hp(a):
        v16 = a.astype(jnp.bfloat16).reshape(N, 4, 2, 16)
        sw = jnp.swapaxes(v16, 2, 3)
        return lax.bitcast_convert_type(sw, jnp.int32).reshape(N, D // 2)

    for i in range(NLAYERS - 1):
        sp = _scat_call(_pack_hp(hp), row3, col3)
        hp = _stage_b(sp, hp, x0, dinv, ws[i], betas[i])
    sp = _scat_call(_pack_hp(hp), row3, col3)
    return _stage_b_final(sp, hp, x0, dinv, ws[-1], Wlast,
                          blast.reshape(1, DOUT), betas[-1])


# speedup vs baseline: 1.0605x; 1.0605x over previous
import math

import jax
import jax.numpy as jnp
from jax import lax
from jax.experimental import pallas as pl
from jax.experimental.pallas import tpu as pltpu
from jax.experimental.pallas import tpu_sc as plsc

N = 10000
E = 320000
D = 128
DOUT = 64
ALPHA = 0.1
THETA = 0.5
NLAYERS = 4

NC = 2
NS = 16
NW = NC * NS
B = 128
C0 = 109
C1 = 49
EPAD = (C0 + C1) * NS * B
NPAD = 10112
RPT = NPAD // NS
DNPAD = 10240
DRPT = DNPAD // NS
BN = 1000

_mesh = plsc.VectorSubcoreMesh(core_axis_name="c", subcore_axis_name="s")


def _deg_body(col_hbm, zeros_hbm, out_hbm, colv, onesv, deg_sh):
    cid = lax.axis_index("c")
    sid = lax.axis_index("s")
    pltpu.sync_copy(zeros_hbm.at[pl.ds(sid * DRPT, DRPT)],
                    deg_sh.at[pl.ds(sid * DRPT, DRPT)])
    pltpu.sync_copy(col_hbm.at[cid, sid], colv)
    for i in range(B // 16):
        onesv[pl.ds(i * 16, 16)] = jnp.full((16,), 1.0, jnp.float32)
    plsc.subcore_barrier()

    def body(j, carry):
        pltpu.sync_copy(onesv, deg_sh.at[colv.at[j]], add=True)
        return carry

    @pl.when(cid == 0)
    def _():
        lax.fori_loop(0, C0, body, 0)

    @pl.when(cid == 1)
    def _():
        lax.fori_loop(0, C1, body, 0)

    plsc.subcore_barrier()
    pltpu.sync_copy(deg_sh.at[pl.ds(sid * DRPT, DRPT)],
                    out_hbm.at[pl.ds(cid * DNPAD + sid * DRPT, DRPT)])


_deg_call = pl.kernel(
    _deg_body,
    out_type=jax.ShapeDtypeStruct((NC * DNPAD,), jnp.float32),
    mesh=_mesh,
    scratch_types=[
        pltpu.VMEM((C0, B), jnp.int32),
        pltpu.VMEM((B,), jnp.float32),
        pltpu.VMEM_SHARED((DNPAD,), jnp.float32),
    ],
)


def _scat_body(hp_hbm, row_hbm, col_hbm, out_hbm,
               ridx, cidx, ibuf, fbuf, s_sh, gsem, ssem, rsem, csem):
    cid = lax.axis_index("c")
    sid = lax.axis_index("s")

    @plsc.parallel_loop(0, B * (D // 16), unroll=8)
    def _zfill(i):
        r = lax.shift_right_logical(i, 3)
        kk = jnp.bitwise_and(i, 7)
        fbuf[0, r, pl.ds(kk * 16, 16)] = jnp.zeros((16,), jnp.float32)

    for q in range(RPT // B):
        pltpu.sync_copy(fbuf.at[0],
                        s_sh.at[pl.ds(sid * RPT + q * B, B)])
    _ztail = RPT - (RPT // B) * B
    if _ztail:
        pltpu.sync_copy(fbuf.at[0, pl.ds(0, _ztail)],
                        s_sh.at[pl.ds(sid * RPT + (RPT // B) * B, _ztail)])
    plsc.subcore_barrier()

    def gather(j, b):
        pltpu.async_copy(hp_hbm.at[ridx.at[b]], ibuf.at[b], gsem)

    def gwait(j, b):
        pltpu.make_async_copy(hp_hbm.at[ridx.at[b]], ibuf.at[b], gsem).wait()

    def rowload(j, s):
        pltpu.async_copy(row_hbm.at[cid, sid, j], ridx.at[s], rsem)

    def rowwait(j, s):
        pltpu.make_async_copy(row_hbm.at[cid, sid, j], ridx.at[s], rsem).wait()

    def colload(j, s):
        pltpu.async_copy(col_hbm.at[cid, sid, j], cidx.at[s], csem)

    def colwait(j, s):
        pltpu.make_async_copy(col_hbm.at[cid, sid, j], cidx.at[s], csem).wait()

    def scat(j, b):
        pltpu.async_copy(fbuf.at[b], s_sh.at[cidx.at[b]], ssem, add=True)

    def swait(j, b):
        pltpu.make_async_copy(fbuf.at[b], s_sh.at[cidx.at[b]], ssem).wait()

    def unpack_chunk(b):
        @plsc.parallel_loop(0, B, unroll=4)
        def urow(r):
            for k in range(4):
                w = ibuf[b, r, pl.ds(k * 16, 16)]
                lo = lax.bitcast_convert_type(
                    jnp.left_shift(w, 16), jnp.float32)
                hi = lax.bitcast_convert_type(
                    jnp.bitwise_and(w, jnp.int32(-65536)), jnp.float32)
                fbuf[b, r, pl.ds(32 * k, 16)] = lo
                fbuf[b, r, pl.ds(32 * k + 16, 16)] = hi

    def pipeline(cn):
        pltpu.sync_copy(row_hbm.at[cid, sid, 0], ridx.at[0])
        gather(0, 0)
        if cn > 1:
            rowload(1, 1)
        colload(0, 0)

        def body(j, carry):
            b = lax.rem(j, 2)
            o = 1 - b

            @pl.when(j > 0)
            def _():
                swait(j - 1, o)

            gwait(j, b)

            @pl.when(j + 1 < cn)
            def _():
                rowwait(j + 1, o)
                gather(j + 1, o)

            @pl.when(j + 2 < cn)
            def _():
                rowload(j + 2, b)

            unpack_chunk(b)
            colwait(j, b)

            @pl.when(j + 1 < cn)
            def _():
                colload(j + 1, o)

            scat(j, b)
            return carry

        lax.fori_loop(0, cn, body, 0)
        last = (cn - 1) % 2
        swait(cn - 1, last)

    @pl.when(cid == 0)
    def _():
        pipeline(C0)

    @pl.when(cid == 1)
    def _():
        pipeline(C1)

    plsc.subcore_barrier()
    pltpu.sync_copy(s_sh.at[pl.ds(sid * RPT, RPT)],
                    out_hbm.at[cid, pl.ds(sid * RPT, RPT)])


_scat_call = pl.kernel(
    _scat_body,
    out_type=jax.ShapeDtypeStruct((NC, NPAD, D), jnp.float32),
    mesh=_mesh,
    compiler_params=pltpu.CompilerParams(use_tc_tiling_on_sc=False),
    scratch_types=[
        pltpu.VMEM((2, B), jnp.int32),
        pltpu.VMEM((2, B), jnp.int32),
        pltpu.VMEM((2, B, D // 2), jnp.int32),
        pltpu.VMEM((2, B, D), jnp.float32),
        pltpu.VMEM_SHARED((NPAD, D), jnp.float32),
        pltpu.SemaphoreType.DMA,
        pltpu.SemaphoreType.DMA,
        pltpu.SemaphoreType.DMA,
        pltpu.SemaphoreType.DMA,
    ],
)


def _pack_rows(t):
    def rne(x):
        u = lax.bitcast_convert_type(x, jnp.int32)
        r = u + 0x7FFF + jnp.bitwise_and(lax.shift_right_logical(u, 16), 1)
        return jnp.bitwise_and(lax.shift_right_logical(r, 16), 0xFFFF)

    words = []
    for g in range(4):
        a = rne(t[:, 32 * g:32 * g + 16])
        b2 = rne(t[:, 32 * g + 16:32 * g + 32])
        words.append(jnp.bitwise_or(a, jnp.left_shift(b2, 16)))
    return jnp.concatenate(words, axis=1)


def _stage_a_body(x_ref, w_ref, b_ref, degt_ref, x0_ref, hp_ref, hq_ref,
                  dinv_ref):
    h = jnp.dot(x_ref[...], w_ref[...], preferred_element_type=jnp.float32)
    h = jnp.maximum(h + b_ref[...], 0.0)
    deg = degt_ref[:, 0:1] + degt_ref[:, 1:2] + 1.0
    dinv = lax.rsqrt(deg)
    x0_ref[...] = h
    hp = dinv * h
    hp_ref[...] = hp
    hq_ref[...] = _pack_rows(hp)
    dinv_ref[...] = dinv


def _stage_a(x, w0, b0, degt):
    return pl.pallas_call(
        _stage_a_body,
        grid=(N // BN,),
        in_specs=[
            pl.BlockSpec((BN, D), lambda i: (i, 0)),
            pl.BlockSpec((D, D), lambda i: (0, 0)),
            pl.BlockSpec((1, D), lambda i: (0, 0)),
            pl.BlockSpec((BN, NC), lambda i: (i, 0)),
        ],
        out_specs=[
            pl.BlockSpec((BN, D), lambda i: (i, 0)),
            pl.BlockSpec((BN, D), lambda i: (i, 0)),
            pl.BlockSpec((BN, D // 2), lambda i: (i, 0)),
            pl.BlockSpec((BN, 1), lambda i: (i, 0)),
        ],
        out_shape=[
            jax.ShapeDtypeStruct((N, D), jnp.float32),
            jax.ShapeDtypeStruct((N, D), jnp.float32),
            jax.ShapeDtypeStruct((N, D // 2), jnp.int32),
            jax.ShapeDtypeStruct((N, 1), jnp.float32),
        ],
    )(x, w0, b0, degt)


def _layer_common(beta, sp_ref, hp_ref, x0_ref, dinv_ref, w_ref):
    s = sp_ref[0] + sp_ref[1] + hp_ref[...]
    agg = dinv_ref[...] * s
    s = (1.0 - ALPHA) * agg + ALPHA * x0_ref[...]
    t = jnp.dot(s, w_ref[...], preferred_element_type=jnp.float32)
    return (1.0 - beta) * s + beta * t


def _stage_b(sp, hp, x0, dinv, w, beta):
    def body(sp_ref, hp_ref, x0_ref, dinv_ref, w_ref, out_ref, outq_ref):
        t = _layer_common(beta, sp_ref, hp_ref, x0_ref, dinv_ref, w_ref)
        hp2 = dinv_ref[...] * jnp.maximum(t, 0.0)
        out_ref[...] = hp2
        outq_ref[...] = _pack_rows(hp2)

    return pl.pallas_call(
        body,
        grid=(N // BN,),
        in_specs=[
            pl.BlockSpec((NC, BN, D), lambda i: (0, i, 0)),
            pl.BlockSpec((BN, D), lambda i: (i, 0)),
            pl.BlockSpec((BN, D), lambda i: (i, 0)),
            pl.BlockSpec((BN, 1), lambda i: (i, 0)),
            pl.BlockSpec((D, D), lambda i: (0, 0)),
        ],
        out_specs=[
            pl.BlockSpec((BN, D), lambda i: (i, 0)),
            pl.BlockSpec((BN, D // 2), lambda i: (i, 0)),
        ],
        out_shape=[
            jax.ShapeDtypeStruct((N, D), jnp.float32),
            jax.ShapeDtypeStruct((N, D // 2), jnp.int32),
        ],
    )(sp, hp, x0, dinv, w)


def _stage_b_final(sp, hp, x0, dinv, w, wlast, blast, beta):
    def body(sp_ref, hp_ref, x0_ref, dinv_ref, w_ref, wl_ref, bl_ref, out_ref):
        t = _layer_common(beta, sp_ref, hp_ref, x0_ref, dinv_ref, w_ref)
        h = jnp.maximum(t, 0.0)
        out_ref[...] = (jnp.dot(h, wl_ref[...], preferred_element_type=jnp.float32)
                        + bl_ref[...])

    return pl.pallas_call(
        body,
        grid=(N // BN,),
        in_specs=[
            pl.BlockSpec((NC, BN, D), lambda i: (0, i, 0)),
            pl.BlockSpec((BN, D), lambda i: (i, 0)),
            pl.BlockSpec((BN, D), lambda i: (i, 0)),
            pl.BlockSpec((BN, 1), lambda i: (i, 0)),
            pl.BlockSpec((D, D), lambda i: (0, 0)),
            pl.BlockSpec((D, DOUT), lambda i: (0, 0)),
            pl.BlockSpec((1, DOUT), lambda i: (0, 0)),
        ],
        out_specs=pl.BlockSpec((BN, DOUT), lambda i: (i, 0)),
        out_shape=jax.ShapeDtypeStruct((N, DOUT), jnp.float32),
    )(sp, hp, x0, dinv, w, wlast, blast)


def kernel(x, edge_index, W0, b0, W1, W2, W3, W4, Wlast, blast):
    row = edge_index[0]
    col = edge_index[1]
    pad = EPAD - E
    e0 = NS * C0 * B
    cpad = NS * C0 * B - (EPAD - e0)

    def _split(a, fill):
        ap = jnp.concatenate([a, jnp.full((pad,), fill, jnp.int32)])
        a0 = ap[:e0].reshape(NS, C0, B)
        a1 = jnp.pad(ap[e0:].reshape(NS, C1, B),
                     ((0, 0), (0, C0 - C1), (0, 0)), constant_values=fill)
        return jnp.stack([a0, a1])

    row3 = _split(row, 0)
    col3 = _split(col, N)
    zeros1 = jnp.zeros((DNPAD,), jnp.float32)

    degp = _deg_call(col3, zeros1).reshape(NC, DNPAD)
    degt = degp[:, :N].T
    x0, hp, hq, dinv = _stage_a(x, W0, b0.reshape(1, D), degt)

    betas = [math.log(THETA / i + 1.0) for i in range(1, NLAYERS + 1)]
    ws = [W1, W2, W3, W4]
    for i in range(NLAYERS - 1):
        sp = _scat_call(hq, row3, col3)
        hp, hq = _stage_b(sp, hp, x0, dinv, ws[i], betas[i])
    sp = _scat_call(hq, row3, col3)
    return _stage_b_final(sp, hp, x0, dinv, ws[-1], Wlast,
                          blast.reshape(1, DOUT), betas[-1])
